# Initial kernel scaffold; baseline (speedup 1.0000x reference)
import functools

import jax
import jax.numpy as jnp
from jax import lax
from jax.experimental import pallas as pl
from jax.experimental.pallas import tpu as pltpu
from jax.experimental.pallas import tpu_sc as plsc

EMBED = 384
NH = 8
NP = 12
NL = 8
HS = 32
WS = 32
HW = HS * WS
DPH = EMBED // NH
HD = DPH // NH
G = 12
IMGS = G * NH
F32 = jnp.float32


def _proj_body(x_ref, xv_ref, pos_ref, w1_ref, b1_ref, w2_ref, b2_ref,
               so_ref, aw_ref, v_ref):
    q = x_ref[0] + pos_ref[0]
    o1 = jnp.dot(w1_ref[0], q, preferred_element_type=F32) + b1_ref[0]
    so_ref[0] = o1[:NH * NP * 2]
    aw_ref[0] = o1[NH * NP * 2:]
    v_ref[0] = jnp.dot(w2_ref[0], xv_ref[0], preferred_element_type=F32) + b2_ref[0]


def _out_body(s_ref, x_ref, w_ref, b_ref, o_ref):
    o_ref[0] = (jnp.dot(w_ref[0], s_ref[0], preferred_element_type=F32)
                + b_ref[0] + x_ref[0])


def _proj_call(x, posT, w1, b1, w2, b2):
    n1 = NH * NP * 2 + NH * NP
    return pl.pallas_call(
        _proj_body,
        grid=(G,),
        in_specs=[
            pl.BlockSpec((1, EMBED, HW), lambda g: (g, 0, 0)),
            pl.BlockSpec((1, EMBED, HW), lambda g: ((g + 6) % 12, 0, 0)),
            pl.BlockSpec((1, EMBED, HW), lambda g: (g // 6, 0, 0)),
            pl.BlockSpec((1, n1, EMBED), lambda g: (g // 6, 0, 0)),
            pl.BlockSpec((1, n1, 1), lambda g: (g // 6, 0, 0)),
            pl.BlockSpec((1, DPH, EMBED), lambda g: (g // 6, 0, 0)),
            pl.BlockSpec((1, DPH, 1), lambda g: (g // 6, 0, 0)),
        ],
        out_specs=[
            pl.BlockSpec((1, NH * NP * 2, HW), lambda g: (g, 0, 0)),
            pl.BlockSpec((1, NH * NP, HW), lambda g: (g, 0, 0)),
            pl.BlockSpec((1, DPH, HW), lambda g: (g, 0, 0)),
        ],
        out_shape=[
            jax.ShapeDtypeStruct((G, NH * NP * 2, HW), F32),
            jax.ShapeDtypeStruct((G, NH * NP, HW), F32),
            jax.ShapeDtypeStruct((G, DPH, HW), F32),
        ],
    )(x, x, posT, w1, b1, w2, b2)


def _out_call(samp, x, w, b):
    return pl.pallas_call(
        _out_body,
        grid=(G,),
        in_specs=[
            pl.BlockSpec((1, DPH, HW), lambda g: (g, 0, 0)),
            pl.BlockSpec((1, EMBED, HW), lambda g: (g, 0, 0)),
            pl.BlockSpec((1, EMBED, DPH), lambda g: (g // 6, 0, 0)),
            pl.BlockSpec((1, EMBED, 1), lambda g: (g // 6, 0, 0)),
        ],
        out_specs=pl.BlockSpec((1, EMBED, HW), lambda g: (g, 0, 0)),
        out_shape=jax.ShapeDtypeStruct((G, EMBED, HW), F32),
    )(samp, x, w, b)


@functools.lru_cache(maxsize=None)
def _sc_sampler():
    info = plsc.get_sparse_core_info()
    NC, NS, L = info.num_cores, info.num_subcores, info.num_lanes
    NW = NC * NS
    per_w = IMGS // NW
    nblk = HW // L
    mesh = plsc.VectorSubcoreMesh(core_axis_name="c", subcore_axis_name="s")

    @functools.partial(
        pl.kernel,
        mesh=mesh,
        out_type=jax.ShapeDtypeStruct((IMGS, HD, HW), F32),
        scratch_types=[
            pltpu.VMEM((HD, HW), F32),
            pltpu.VMEM((NP * 2, HW), F32),
            pltpu.VMEM((NP, HW), F32),
            pltpu.VMEM((HD, HW), F32),
        ],
    )
    def _sc_sample(v_hbm, so_hbm, aw_hbm, out_hbm, v_v, so_v, aw_v, o_v):
        wid = lax.axis_index("s") * NC + lax.axis_index("c")
        lane = lax.iota(jnp.int32, L)

        def img_body(k, carry):
            m = wid * per_w + k
            pltpu.sync_copy(v_hbm.at[m], v_v)
            pltpu.sync_copy(so_hbm.at[m], so_v)
            pltpu.sync_copy(aw_hbm.at[m], aw_v)

            def blk(i, carry2):
                base = i * L
                rowf = (base // WS).astype(F32)
                colf = ((base % WS) + lane).astype(F32)
                z = [aw_v[p, pl.ds(base, L)] for p in range(NP)]
                mx = z[0]
                for p in range(1, NP):
                    mx = jnp.maximum(mx, z[p])
                e = [jnp.exp(zp - mx) for zp in z]
                ssum = e[0]
                for p in range(1, NP):
                    ssum = ssum + e[p]
                acc = [jnp.zeros((L,), F32) for _ in range(HD)]
                for p in range(NP):
                    x = colf + so_v[2 * p, pl.ds(base, L)]
                    y = rowf + so_v[2 * p + 1, pl.ds(base, L)]
                    xt = x.astype(jnp.int32)
                    x0 = jnp.where(xt.astype(F32) > x, xt - 1, xt)
                    fx1 = x - x0.astype(F32)
                    fx0 = 1.0 - fx1
                    yt = y.astype(jnp.int32)
                    y0 = jnp.where(yt.astype(F32) > y, yt - 1, yt)
                    fy1 = y - y0.astype(F32)
                    fy0 = 1.0 - fy1
                    for dx, dy, wx, wy in ((0, 0, fx0, fy0), (1, 0, fx1, fy0),
                                           (0, 1, fx0, fy1), (1, 1, fx1, fy1)):
                        xi = x0 + dx
                        yi = y0 + dy
                        ok = (xi >= 0) & (xi < WS) & (yi >= 0) & (yi < HS)
                        xc = jnp.minimum(jnp.maximum(xi, 0), WS - 1)
                        yc = jnp.minimum(jnp.maximum(yi, 0), HS - 1)
                        pix = yc * WS + xc
                        wt = jnp.where(ok, wx * wy * e[p], 0.0)
                        for c in range(HD):
                            cv = jnp.full((L,), c, jnp.int32)
                            val = plsc.load_gather(v_v, [cv, pix])
                            acc[c] = acc[c] + wt * val
                rs = 1.0 / ssum
                for c in range(HD):
                    o_v[c, pl.ds(base, L)] = acc[c] * rs
                return carry2

            lax.fori_loop(0, nblk, blk, 0)
            pltpu.sync_copy(o_v, out_hbm.at[m])
            return carry

        lax.fori_loop(0, per_w, img_body, 0)

    return _sc_sample


def _posT_one(re_s, ce_s):
    pe = jnp.concatenate([
        jnp.broadcast_to(ce_s[None, :, :], (HS, WS, EMBED // 2)),
        jnp.broadcast_to(re_s[:, None, :], (HS, WS, EMBED // 2))], -1)
    return pe.reshape(HW, EMBED).T


def kernel(rgb_fea, ir_fea, so_W, so_b, aw_W, aw_b, vp_W, vp_b, op_W, op_b,
           row_embed, col_embed):
    xr = rgb_fea.transpose(0, 2, 1, 3, 4).reshape(6, EMBED, HW)
    xi = ir_fea.transpose(0, 2, 1, 3, 4).reshape(6, EMBED, HW)
    x = jnp.concatenate([xr, xi], axis=0)

    posT = jnp.stack([_posT_one(row_embed[0], col_embed[0]),
                      _posT_one(row_embed[1], col_embed[1])])

    w1 = jnp.swapaxes(jnp.concatenate([so_W, aw_W], -1), -1, -2)
    b1 = jnp.concatenate([so_b, aw_b], -1)[..., None]
    w2 = jnp.swapaxes(vp_W, -1, -2)
    b2 = vp_b[..., None]
    w3 = jnp.swapaxes(op_W, -1, -2)
    b3 = op_b[..., None]

    sampler = _sc_sampler()
    for l in range(NL):
        so_t, aw_t, v_t = _proj_call(x, posT, w1[:, l], b1[:, l],
                                     w2[:, l], b2[:, l])
        samp = sampler(v_t.reshape(IMGS, HD, HW),
                       so_t.reshape(IMGS, NP * 2, HW),
                       aw_t.reshape(IMGS, NP, HW))
        x = _out_call(samp.reshape(G, DPH, HW), x, w3[:, l], b3[:, l])

    y = x.reshape(2, 2, 3, EMBED, HS, WS).transpose(0, 1, 3, 2, 4, 5)
    return y


# speedup vs baseline: 530.9426x; 530.9426x over previous
import functools

import jax
import jax.numpy as jnp
from jax import lax
from jax.experimental import pallas as pl
from jax.experimental.pallas import tpu as pltpu
from jax.experimental.pallas import tpu_sc as plsc

EMBED = 384
NH = 8
NP = 12
NL = 8
HS = 32
WS = 32
HW = HS * WS
DPH = EMBED // NH
HD = DPH // NH
G = 12
IMGS = G * NH
F32 = jnp.float32


def _proj_body(x_ref, xv_ref, pos_ref, w1_ref, b1_ref, w2_ref, b2_ref,
               so_ref, aw_ref, v_ref):
    q = x_ref[0] + pos_ref[0]
    o1 = jnp.dot(w1_ref[0], q, preferred_element_type=F32) + b1_ref[0]
    so_ref[0] = o1[:NH * NP * 2]
    aw_ref[0] = o1[NH * NP * 2:]
    v_ref[0] = jnp.dot(w2_ref[0], xv_ref[0], preferred_element_type=F32) + b2_ref[0]


def _out_body(s_ref, x_ref, w_ref, b_ref, o_ref):
    o_ref[0] = (jnp.dot(w_ref[0], s_ref[0], preferred_element_type=F32)
                + b_ref[0] + x_ref[0])


def _proj_call(x, posT, w1, b1, w2, b2):
    n1 = NH * NP * 2 + NH * NP
    return pl.pallas_call(
        _proj_body,
        grid=(G,),
        in_specs=[
            pl.BlockSpec((1, EMBED, HW), lambda g: (g, 0, 0)),
            pl.BlockSpec((1, EMBED, HW), lambda g: ((g + 6) % 12, 0, 0)),
            pl.BlockSpec((1, EMBED, HW), lambda g: (g // 6, 0, 0)),
            pl.BlockSpec((1, n1, EMBED), lambda g: (g // 6, 0, 0)),
            pl.BlockSpec((1, n1, 1), lambda g: (g // 6, 0, 0)),
            pl.BlockSpec((1, DPH, EMBED), lambda g: (g // 6, 0, 0)),
            pl.BlockSpec((1, DPH, 1), lambda g: (g // 6, 0, 0)),
        ],
        out_specs=[
            pl.BlockSpec((1, NH * NP * 2, HW), lambda g: (g, 0, 0)),
            pl.BlockSpec((1, NH * NP, HW), lambda g: (g, 0, 0)),
            pl.BlockSpec((1, DPH, HW), lambda g: (g, 0, 0)),
        ],
        out_shape=[
            jax.ShapeDtypeStruct((G, NH * NP * 2, HW), F32),
            jax.ShapeDtypeStruct((G, NH * NP, HW), F32),
            jax.ShapeDtypeStruct((G, DPH, HW), F32),
        ],
    )(x, x, posT, w1, b1, w2, b2)


def _out_call(samp, x, w, b):
    return pl.pallas_call(
        _out_body,
        grid=(G,),
        in_specs=[
            pl.BlockSpec((1, DPH, HW), lambda g: (g, 0, 0)),
            pl.BlockSpec((1, EMBED, HW), lambda g: (g, 0, 0)),
            pl.BlockSpec((1, EMBED, DPH), lambda g: (g // 6, 0, 0)),
            pl.BlockSpec((1, EMBED, 1), lambda g: (g // 6, 0, 0)),
        ],
        out_specs=pl.BlockSpec((1, EMBED, HW), lambda g: (g, 0, 0)),
        out_shape=jax.ShapeDtypeStruct((G, EMBED, HW), F32),
    )(samp, x, w, b)


@functools.lru_cache(maxsize=None)
def _sc_sampler():
    info = plsc.get_sparse_core_info()
    NC, NS, L = info.num_cores, info.num_subcores, info.num_lanes
    NW = NC * NS
    per_w = IMGS // NW
    nblk = HW // L
    mesh = plsc.VectorSubcoreMesh(core_axis_name="c", subcore_axis_name="s")

    @functools.partial(
        pl.kernel,
        mesh=mesh,
        out_type=jax.ShapeDtypeStruct((IMGS, HD * HW), F32),
        compiler_params=pltpu.CompilerParams(needs_layout_passes=False),
        scratch_types=[
            pltpu.VMEM((HD * HW,), F32),
            pltpu.VMEM((NP * 2, HW), F32),
            pltpu.VMEM((NP, HW), F32),
            pltpu.VMEM((HD * HW,), F32),
        ],
    )
    def _sc_sample(v_hbm, so_hbm, aw_hbm, out_hbm, v_v, so_v, aw_v, o_v):
        wid = lax.axis_index("s") * NC + lax.axis_index("c")
        lane = lax.iota(jnp.int32, L)

        def img_body(k, carry):
            m = wid * per_w + k
            pltpu.sync_copy(v_hbm.at[m], v_v)
            pltpu.sync_copy(so_hbm.at[m], so_v)
            pltpu.sync_copy(aw_hbm.at[m], aw_v)

            def blk(i, carry2):
                base = i * L
                rowf = (base // WS).astype(F32)
                colf = ((base % WS) + lane).astype(F32)
                z = [aw_v[p, pl.ds(base, L)] for p in range(NP)]
                mx = z[0]
                for p in range(1, NP):
                    mx = jnp.maximum(mx, z[p])
                e = [jnp.exp(zp - mx) for zp in z]
                ssum = e[0]
                for p in range(1, NP):
                    ssum = ssum + e[p]
                acc = [jnp.zeros((L,), F32) for _ in range(HD)]
                for p in range(NP):
                    x = colf + so_v[2 * p, pl.ds(base, L)]
                    y = rowf + so_v[2 * p + 1, pl.ds(base, L)]
                    xt = x.astype(jnp.int32)
                    x0 = jnp.where(xt.astype(F32) > x, xt - 1, xt)
                    fx1 = x - x0.astype(F32)
                    fx0 = 1.0 - fx1
                    yt = y.astype(jnp.int32)
                    y0 = jnp.where(yt.astype(F32) > y, yt - 1, yt)
                    fy1 = y - y0.astype(F32)
                    fy0 = 1.0 - fy1
                    for dx, dy, wx, wy in ((0, 0, fx0, fy0), (1, 0, fx1, fy0),
                                           (0, 1, fx0, fy1), (1, 1, fx1, fy1)):
                        xi = x0 + dx
                        yi = y0 + dy
                        ok = (xi >= 0) & (xi < WS) & (yi >= 0) & (yi < HS)
                        xc = jnp.minimum(jnp.maximum(xi, 0), WS - 1)
                        yc = jnp.minimum(jnp.maximum(yi, 0), HS - 1)
                        pix = yc * WS + xc
                        wt = jnp.where(ok, wx * wy * e[p], 0.0)
                        for c in range(HD):
                            val = plsc.load_gather(v_v, [pix + (c * HW)])
                            acc[c] = acc[c] + wt * val
                rs = 1.0 / ssum
                for c in range(HD):
                    o_v[pl.ds(c * HW + base, L)] = acc[c] * rs
                return carry2

            lax.fori_loop(0, nblk, blk, 0)
            pltpu.sync_copy(o_v, out_hbm.at[m])
            return carry

        lax.fori_loop(0, per_w, img_body, 0)

    return _sc_sample


def _posT_one(re_s, ce_s):
    pe = jnp.concatenate([
        jnp.broadcast_to(ce_s[None, :, :], (HS, WS, EMBED // 2)),
        jnp.broadcast_to(re_s[:, None, :], (HS, WS, EMBED // 2))], -1)
    return pe.reshape(HW, EMBED).T


def kernel(rgb_fea, ir_fea, so_W, so_b, aw_W, aw_b, vp_W, vp_b, op_W, op_b,
           row_embed, col_embed):
    xr = rgb_fea.transpose(0, 2, 1, 3, 4).reshape(6, EMBED, HW)
    xi = ir_fea.transpose(0, 2, 1, 3, 4).reshape(6, EMBED, HW)
    x = jnp.concatenate([xr, xi], axis=0)

    posT = jnp.stack([_posT_one(row_embed[0], col_embed[0]),
                      _posT_one(row_embed[1], col_embed[1])])

    w1 = jnp.swapaxes(jnp.concatenate([so_W, aw_W], -1), -1, -2)
    b1 = jnp.concatenate([so_b, aw_b], -1)[..., None]
    w2 = jnp.swapaxes(vp_W, -1, -2)
    b2 = vp_b[..., None]
    w3 = jnp.swapaxes(op_W, -1, -2)
    b3 = op_b[..., None]

    sampler = _sc_sampler()
    for l in range(NL):
        so_t, aw_t, v_t = _proj_call(x, posT, w1[:, l], b1[:, l],
                                     w2[:, l], b2[:, l])
        samp = sampler(v_t.reshape(IMGS, HD * HW),
                       so_t.reshape(IMGS, NP * 2, HW),
                       aw_t.reshape(IMGS, NP, HW))
        x = _out_call(samp.reshape(G, DPH, HW), x, w3[:, l], b3[:, l])

    y = x.reshape(2, 2, 3, EMBED, HS, WS).transpose(0, 1, 3, 2, 4, 5)
    return y


# speedup vs baseline: 631.1873x; 1.1888x over previous
import functools

import jax
import jax.numpy as jnp
from jax import lax
from jax.experimental import pallas as pl
from jax.experimental.pallas import tpu as pltpu
from jax.experimental.pallas import tpu_sc as plsc

EMBED = 384
NH = 8
NP = 12
NL = 8
HS = 32
WS = 32
HW = HS * WS
DPH = EMBED // NH
HD = DPH // NH
G = 12
IMGS = G * NH
F32 = jnp.float32


def _proj_body(x_ref, xv_ref, pos_ref, w1_ref, b1_ref, w2_ref, b2_ref,
               so_ref, aw_ref, v_ref):
    q = x_ref[0] + pos_ref[0]
    o1 = jnp.dot(w1_ref[0], q, preferred_element_type=F32) + b1_ref[0]
    so_ref[0] = o1[:NH * NP * 2]
    a = o1[NH * NP * 2:]
    a = a - jnp.max(a, axis=0, keepdims=True)
    ex = jnp.exp(a)
    hsel = (lax.broadcasted_iota(jnp.int32, (NH, NH * NP), 1) // NP
            == lax.broadcasted_iota(jnp.int32, (NH, NH * NP), 0)).astype(F32)
    rinv = 1.0 / jnp.dot(hsel, ex, preferred_element_type=F32)
    aw_ref[0] = ex * jnp.dot(hsel.T, rinv, preferred_element_type=F32)
    v_ref[0] = jnp.dot(w2_ref[0], xv_ref[0], preferred_element_type=F32) + b2_ref[0]


def _out_body(s_ref, x_ref, w_ref, b_ref, o_ref):
    o_ref[0] = (jnp.dot(w_ref[0], s_ref[0], preferred_element_type=F32)
                + b_ref[0] + x_ref[0])


def _proj_call(x, posT, w1, b1, w2, b2):
    n1 = NH * NP * 2 + NH * NP
    return pl.pallas_call(
        _proj_body,
        grid=(G,),
        in_specs=[
            pl.BlockSpec((1, EMBED, HW), lambda g: (g, 0, 0)),
            pl.BlockSpec((1, EMBED, HW), lambda g: ((g + 6) % 12, 0, 0)),
            pl.BlockSpec((1, EMBED, HW), lambda g: (g // 6, 0, 0)),
            pl.BlockSpec((1, n1, EMBED), lambda g: (g // 6, 0, 0)),
            pl.BlockSpec((1, n1, 1), lambda g: (g // 6, 0, 0)),
            pl.BlockSpec((1, DPH, EMBED), lambda g: (g // 6, 0, 0)),
            pl.BlockSpec((1, DPH, 1), lambda g: (g // 6, 0, 0)),
        ],
        out_specs=[
            pl.BlockSpec((1, NH * NP * 2, HW), lambda g: (g, 0, 0)),
            pl.BlockSpec((1, NH * NP, HW), lambda g: (g, 0, 0)),
            pl.BlockSpec((1, DPH, HW), lambda g: (g, 0, 0)),
        ],
        out_shape=[
            jax.ShapeDtypeStruct((G, NH * NP * 2, HW), F32),
            jax.ShapeDtypeStruct((G, NH * NP, HW), F32),
            jax.ShapeDtypeStruct((G, DPH, HW), F32),
        ],
    )(x, x, posT, w1, b1, w2, b2)


def _out_call(samp, x, w, b):
    return pl.pallas_call(
        _out_body,
        grid=(G,),
        in_specs=[
            pl.BlockSpec((1, DPH, HW), lambda g: (g, 0, 0)),
            pl.BlockSpec((1, EMBED, HW), lambda g: (g, 0, 0)),
            pl.BlockSpec((1, EMBED, DPH), lambda g: (g // 6, 0, 0)),
            pl.BlockSpec((1, EMBED, 1), lambda g: (g // 6, 0, 0)),
        ],
        out_specs=pl.BlockSpec((1, EMBED, HW), lambda g: (g, 0, 0)),
        out_shape=jax.ShapeDtypeStruct((G, EMBED, HW), F32),
    )(samp, x, w, b)


PW = 36
PIMG = PW * PW
PTAB = HD * PIMG


@functools.lru_cache(maxsize=None)
def _sc_sampler():
    info = plsc.get_sparse_core_info()
    NC, NS, L = info.num_cores, info.num_subcores, info.num_lanes
    NW = NC * NS
    per_w = IMGS // NW
    nblk = HW // L
    mesh = plsc.VectorSubcoreMesh(core_axis_name="c", subcore_axis_name="s")

    @functools.partial(
        pl.kernel,
        mesh=mesh,
        out_type=jax.ShapeDtypeStruct((IMGS, HD * HW), F32),
        compiler_params=pltpu.CompilerParams(needs_layout_passes=False,
                                             disable_bounds_checks=True),
        scratch_types=[
            pltpu.VMEM((PTAB,), F32),
            pltpu.VMEM((PTAB,), F32),
            pltpu.VMEM((NP * 2, HW), F32),
            pltpu.VMEM((NP * 2, HW), F32),
            pltpu.VMEM((NP, HW), F32),
            pltpu.VMEM((NP, HW), F32),
            pltpu.VMEM((HD * HW,), F32),
            pltpu.SemaphoreType.DMA,
            pltpu.SemaphoreType.DMA,
        ],
    )
    def _sc_sample(v_hbm, so_hbm, aw_hbm, out_hbm,
                   v_0, v_1, so_0, so_1, aw_0, aw_1, o_v, sem0, sem1):
        wid = lax.axis_index("s") * NC + lax.axis_index("c")
        lane = lax.iota(jnp.int32, L)
        bufs = ((v_0, so_0, aw_0, sem0), (v_1, so_1, aw_1, sem1))

        def start(k):
            m = wid * per_w + k
            vb, sb, ab, sem = bufs[k % 2]
            return (pltpu.async_copy(v_hbm.at[m], vb, sem),
                    pltpu.async_copy(so_hbm.at[m], sb, sem),
                    pltpu.async_copy(aw_hbm.at[m], ab, sem))

        pend = {0: start(0)}
        for k in range(per_w):
            v_v, so_v, aw_v, _ = bufs[k % 2]
            if k + 1 < per_w:
                pend[k + 1] = start(k + 1)
            for h in pend.pop(k):
                h.wait()
            taps = [v_v.at[pl.ds(c * PIMG, PIMG)] for c in range(HD)]

            def blk(i, carry2):
                base = i * L
                rowf = (base // WS).astype(F32)
                colf = ((base % WS) + lane).astype(F32)
                acc = [jnp.zeros((L,), F32) for _ in range(HD)]
                for p in range(NP):
                    ep = aw_v[p, pl.ds(base, L)]
                    xs = jnp.minimum(jnp.maximum(
                        colf + so_v[2 * p, pl.ds(base, L)] + 1.0, 0.0), 33.0)
                    ys = jnp.minimum(jnp.maximum(
                        rowf + so_v[2 * p + 1, pl.ds(base, L)] + 1.0, 0.0), 33.0)
                    x0i = xs.astype(jnp.int32)
                    y0i = ys.astype(jnp.int32)
                    fx1 = xs - x0i.astype(F32)
                    fx0 = 1.0 - fx1
                    fy1 = ys - y0i.astype(F32)
                    fy0 = 1.0 - fy1
                    pix = y0i * PW + x0i
                    idxs = (pix, pix + 1, pix + PW, pix + (PW + 1))
                    ex0 = fx0 * ep
                    ex1 = fx1 * ep
                    wts = (ex0 * fy0, ex1 * fy0, ex0 * fy1, ex1 * fy1)
                    for c in range(HD):
                        a = acc[c]
                        for t in range(4):
                            a = a + wts[t] * plsc.load_gather(taps[c], [idxs[t]])
                        acc[c] = a
                for c in range(HD):
                    o_v[pl.ds(c * HW + base, L)] = acc[c]
                return carry2

            lax.fori_loop(0, nblk, blk, 0)
            pltpu.sync_copy(o_v, out_hbm.at[wid * per_w + k])

    return _sc_sample


def _posT_one(re_s, ce_s):
    pe = jnp.concatenate([
        jnp.broadcast_to(ce_s[None, :, :], (HS, WS, EMBED // 2)),
        jnp.broadcast_to(re_s[:, None, :], (HS, WS, EMBED // 2))], -1)
    return pe.reshape(HW, EMBED).T


def kernel(rgb_fea, ir_fea, so_W, so_b, aw_W, aw_b, vp_W, vp_b, op_W, op_b,
           row_embed, col_embed):
    xr = rgb_fea.transpose(0, 2, 1, 3, 4).reshape(6, EMBED, HW)
    xi = ir_fea.transpose(0, 2, 1, 3, 4).reshape(6, EMBED, HW)
    x = jnp.concatenate([xr, xi], axis=0)

    posT = jnp.stack([_posT_one(row_embed[0], col_embed[0]),
                      _posT_one(row_embed[1], col_embed[1])])

    w1 = jnp.swapaxes(jnp.concatenate([so_W, aw_W], -1), -1, -2)
    b1 = jnp.concatenate([so_b, aw_b], -1)[..., None]
    w2 = jnp.swapaxes(vp_W, -1, -2)
    b2 = vp_b[..., None]
    w3 = jnp.swapaxes(op_W, -1, -2)
    b3 = op_b[..., None]

    sampler = _sc_sampler()
    for l in range(NL):
        so_t, aw_t, v_t = _proj_call(x, posT, w1[:, l], b1[:, l],
                                     w2[:, l], b2[:, l])
        v_pad = jnp.pad(v_t.reshape(IMGS, HD, HS, WS),
                        ((0, 0), (0, 0), (1, 3), (1, 3))).reshape(IMGS, PTAB)
        samp = sampler(v_pad,
                       so_t.reshape(IMGS, NP * 2, HW),
                       aw_t.reshape(IMGS, NP, HW))
        x = _out_call(samp.reshape(G, DPH, HW), x, w3[:, l], b3[:, l])

    y = x.reshape(2, 2, 3, EMBED, HS, WS).transpose(0, 1, 3, 2, 4, 5)
    return y
